# baseline (speedup 1.0000x reference)
import jax
import jax.numpy as jnp
from jax.experimental import pallas as pl

_HID = 32
_K = 30
_G = 64
_D = 97
_BLK = 1000


def _mm0_body(x_ref, w_ref, p_ref):
    p_ref[:] = jnp.dot(x_ref[:], w_ref[:], preferred_element_type=jnp.float32)


def _fused_body(x_ref, b_ref, w_ref, t_ref, p_ref):
    t = jnp.tanh(x_ref[:] + b_ref[:])
    t_ref[:] = t
    p_ref[:] = jnp.dot(t, w_ref[:], preferred_element_type=jnp.float32)


def _tanh_body(x_ref, b_ref, t_ref):
    t_ref[:] = jnp.tanh(x_ref[:] + b_ref[:])


def _mm0(x, w):
    n, din = x.shape
    dout = w.shape[1]
    grid = n // _BLK
    return pl.pallas_call(
        _mm0_body,
        grid=(grid,),
        in_specs=[
            pl.BlockSpec((_BLK, din), lambda i: (i, 0)),
            pl.BlockSpec((din, dout), lambda i: (0, 0)),
        ],
        out_specs=pl.BlockSpec((_BLK, dout), lambda i: (i, 0)),
        out_shape=jax.ShapeDtypeStruct((n, dout), jnp.float32),
    )(x, w)


def _fused(agg, b, w):
    n, din = agg.shape
    dout = w.shape[1]
    grid = n // _BLK
    return pl.pallas_call(
        _fused_body,
        grid=(grid,),
        in_specs=[
            pl.BlockSpec((_BLK, din), lambda i: (i, 0)),
            pl.BlockSpec((1, din), lambda i: (0, 0)),
            pl.BlockSpec((din, dout), lambda i: (0, 0)),
        ],
        out_specs=[
            pl.BlockSpec((_BLK, din), lambda i: (i, 0)),
            pl.BlockSpec((_BLK, dout), lambda i: (i, 0)),
        ],
        out_shape=[
            jax.ShapeDtypeStruct((n, din), jnp.float32),
            jax.ShapeDtypeStruct((n, dout), jnp.float32),
        ],
    )(agg, b.reshape(1, din), w)


def _tanh_bias(agg, b):
    n, din = agg.shape
    grid = n // _BLK
    return pl.pallas_call(
        _tanh_body,
        grid=(grid,),
        in_specs=[
            pl.BlockSpec((_BLK, din), lambda i: (i, 0)),
            pl.BlockSpec((1, din), lambda i: (0, 0)),
        ],
        out_specs=pl.BlockSpec((_BLK, din), lambda i: (i, 0)),
        out_shape=jax.ShapeDtypeStruct((n, din), jnp.float32),
    )(agg, b.reshape(1, din))


def _head_body(x_ref, w1_ref, b1_ref, w2_ref, b2_ref, l1_ref, lb1_ref,
               l2_ref, lb2_ref, o_ref):
    x = x_ref[:]
    w1 = w1_ref[:]
    b1 = b1_ref[:]
    ys = []
    for k in range(_K):
        xk = x[:, k * _D:(k + 1) * _D]
        ys.append(jax.nn.relu(
            jnp.dot(xk, w1, preferred_element_type=jnp.float32) + b1))
    ps = [jnp.maximum(ys[2 * i], ys[2 * i + 1]) for i in range(_K // 2)]
    w2 = w2_ref[:]
    b2 = b2_ref[:]
    zs = []
    for j in range(11):
        win = jnp.concatenate([ps[j + t] for t in range(5)], axis=1)
        zs.append(jax.nn.relu(
            jnp.dot(win, w2, preferred_element_type=jnp.float32) + b2))
    g = jnp.concatenate(zs, axis=1)
    g = jax.nn.relu(
        jnp.dot(g, l1_ref[:], preferred_element_type=jnp.float32) + lb1_ref[:])
    o_ref[:] = jnp.dot(g, l2_ref[:], preferred_element_type=jnp.float32) + lb2_ref[:]


def _head(dense, conv1_w, conv1_b, conv2_w, conv2_b, lin1_w, lin1_b,
          lin2_w, lin2_b):
    w1 = conv1_w[:, 0, :].T
    w2 = jnp.transpose(conv2_w, (2, 1, 0)).reshape(5 * 16, 32)
    l1 = lin1_w.T.reshape(32, 11, 128).transpose(1, 0, 2).reshape(352, 128)
    full = lambda *_: (0, 0)
    return pl.pallas_call(
        _head_body,
        grid=(1,),
        in_specs=[
            pl.BlockSpec((_G, _K * _D), full),
            pl.BlockSpec((_D, 16), full),
            pl.BlockSpec((1, 16), full),
            pl.BlockSpec((80, 32), full),
            pl.BlockSpec((1, 32), full),
            pl.BlockSpec((352, 128), full),
            pl.BlockSpec((1, 128), full),
            pl.BlockSpec((128, 1), full),
            pl.BlockSpec((1, 1), full),
        ],
        out_specs=pl.BlockSpec((_G, 1), full),
        out_shape=jax.ShapeDtypeStruct((_G, 1), jnp.float32),
    )(dense, w1, conv1_b.reshape(1, 16), w2, conv2_b.reshape(1, 32),
      l1, lin1_b.reshape(1, 128), lin2_w.T, lin2_b.reshape(1, 1))


def kernel(z, edge_index, batch, z_table, W0, b0, W1, b1, W2, b2, W3, b3,
           conv1_w, conv1_b, conv2_w, conv2_b, lin1_w, lin1_b, lin2_w, lin2_b):
    n = z.shape[0]
    loop = jnp.arange(n)
    src2 = jnp.concatenate([edge_index[0], loop])
    dst2 = jnp.concatenate([edge_index[1], loop])
    deg = jnp.zeros((n,), jnp.float32).at[dst2].add(1.0)
    dinv = jnp.where(deg > 0, 1.0 / jnp.sqrt(deg), 0.0)
    norm = dinv[src2] * dinv[dst2]

    def agg(p):
        return jnp.zeros((n, p.shape[1]), jnp.float32).at[dst2].add(
            norm[:, None] * p[src2])

    h0 = z_table[z]
    p = _mm0(h0, W0)
    a1 = agg(p)
    t1, p = _fused(a1, b0, W1)
    a2 = agg(p)
    t2, p = _fused(a2, b1, W2)
    a3 = agg(p)
    W3p = jnp.pad(W3, ((0, 0), (0, 127)))
    t3, p = _fused(a3, b2, W3p)
    a4 = agg(p[:, :1])
    t4 = _tanh_bias(a4, b3)

    feat = jnp.concatenate([t1, t2, t3, t4], axis=-1)

    counts = jnp.bincount(batch, length=_G)
    starts = jnp.cumsum(counts) - counts
    order = jnp.lexsort((-feat[:, -1], batch))
    sorted_feat = feat[order]
    k_idx = jnp.arange(_K)
    idx = starts[:, None] + k_idx[None, :]
    gathered = sorted_feat[jnp.clip(idx, 0, n - 1)]
    valid = k_idx[None, :] < counts[:, None]
    dense = jnp.where(valid[:, :, None], gathered, jnp.zeros((), jnp.float32))
    dense = dense.reshape(_G, _K * _D)

    return _head(dense, conv1_w, conv1_b, conv2_w, conv2_b,
                 lin1_w, lin1_b, lin2_w, lin2_b)
